# baseline (speedup 1.0000x reference)
import functools
import jax
import jax.numpy as jnp
from jax import lax
from jax.experimental import pallas as pl
from jax.experimental.pallas import tpu as pltpu
from jax.experimental.pallas import tpu_sc as plsc

_MAXD = 128
_H = 16
_S = 2048
_E = 4104
_NROW = 1024
_LANES = 16


def _sc_body(table_hbm, out_hbm, tab_v, e8_v, sem):
    cid = lax.axis_index("c")
    sid = lax.axis_index("s")
    wid = sid * 2 + cid
    h = wid // 2
    base = (wid % 2) * _NROW

    pltpu.sync_copy(table_hbm, tab_v)

    hvec = jnp.full((_LANES,), h, jnp.int32)
    lane = lax.broadcasted_iota(jnp.int32, (_LANES,), 0)

    def build(c, carry):
        p = c * _LANES + lane
        k = jnp.clip(p - (_S - 1), -_MAXD, _MAXD) + _MAXD
        val = plsc.load_gather(tab_v, [k * _H + hvec])
        msk = p <= 2 * (_S - 1)
        for r in range(8):
            plsc.store_scatter(e8_v, [r * _E + p + r], val, mask=msk)
        return carry

    lax.fori_loop(0, _E // _LANES, build, 0)

    def emit(q, carry):
        i0 = base + q * 8
        copies = []
        for b in range(8):
            i = i0 + b
            r = (b + 1) % 8
            src_off = pl.multiple_of(r * _E + (_S - 1) - i + r, 8)
            dst_off = pl.multiple_of((h * _S + i) * _S, 8)
            copies.append(pltpu.async_copy(
                e8_v.at[pl.ds(src_off, _S)],
                out_hbm.at[pl.ds(dst_off, _S)], sem))
        for c in copies:
            c.wait()
        return carry

    lax.fori_loop(0, _NROW // 8, emit, 0)


def kernel(seq_len, table):
    tab = jnp.zeros((264 * _H,), jnp.float32).at[:(2 * _MAXD + 1) * _H].set(
        table.astype(jnp.float32).reshape(-1))
    mesh = plsc.VectorSubcoreMesh(core_axis_name="c", subcore_axis_name="s")
    run = functools.partial(
        pl.kernel,
        mesh=mesh,
        out_type=jax.ShapeDtypeStruct((_H * _S * _S,), jnp.float32),
        scratch_types=[
            pltpu.VMEM((264 * _H,), jnp.float32),
            pltpu.VMEM((8 * _E,), jnp.float32),
            pltpu.SemaphoreType.DMA,
        ],
        compiler_params=pltpu.CompilerParams(needs_layout_passes=False),
    )(_sc_body)
    return run(tab).reshape(1, _H, _S, _S)


# speedup vs baseline: 1.0114x; 1.0114x over previous
import functools
import jax
import jax.numpy as jnp
from jax import lax
from jax.experimental import pallas as pl
from jax.experimental.pallas import tpu as pltpu
from jax.experimental.pallas import tpu_sc as plsc

_MAXD = 128
_H = 16
_S = 2048
_E = 4104
_NROW = 1024
_LANES = 16


def _sc_body(table_hbm, out_hbm, tab_v, e8_v, sem):
    cid = lax.axis_index("c")
    sid = lax.axis_index("s")
    wid = sid * 2 + cid
    h = wid // 2
    base = (wid % 2) * _NROW

    pltpu.sync_copy(table_hbm, tab_v)

    hvec = jnp.full((_LANES,), h, jnp.int32)
    lane = lax.broadcasted_iota(jnp.int32, (_LANES,), 0)

    def build(c, carry):
        p = c * _LANES + lane
        k = jnp.clip(p - (_S - 1), -_MAXD, _MAXD) + _MAXD
        val = plsc.load_gather(tab_v, [k * _H + hvec])
        msk = p <= 2 * (_S - 1)
        for r in range(8):
            plsc.store_scatter(e8_v, [r * _E + p + r], val, mask=msk)
        return carry

    lax.fori_loop(0, _E // _LANES, build, 0)

    def emit(q, carry):
        i0 = base + q * 16
        copies = []
        for b in range(16):
            i = i0 + b
            r = (b + 1) % 8
            src_off = pl.multiple_of(r * _E + (_S - 1) - i + r, 8)
            dst_off = pl.multiple_of((h * _S + i) * _S, 8)
            copies.append(pltpu.async_copy(
                e8_v.at[pl.ds(src_off, _S)],
                out_hbm.at[pl.ds(dst_off, _S)], sem))
        for c in copies:
            c.wait()
        return carry

    lax.fori_loop(0, _NROW // 16, emit, 0)


def kernel(seq_len, table):
    tab = jnp.zeros((264 * _H,), jnp.float32).at[:(2 * _MAXD + 1) * _H].set(
        table.astype(jnp.float32).reshape(-1))
    mesh = plsc.VectorSubcoreMesh(core_axis_name="c", subcore_axis_name="s")
    run = functools.partial(
        pl.kernel,
        mesh=mesh,
        out_type=jax.ShapeDtypeStruct((_H * _S * _S,), jnp.float32),
        scratch_types=[
            pltpu.VMEM((264 * _H,), jnp.float32),
            pltpu.VMEM((8 * _E,), jnp.float32),
            pltpu.SemaphoreType.DMA,
        ],
        compiler_params=pltpu.CompilerParams(needs_layout_passes=False),
    )(_sc_body)
    return run(tab).reshape(1, _H, _S, _S)


# speedup vs baseline: 4.3998x; 4.3502x over previous
import jax
import jax.numpy as jnp
from jax.experimental import pallas as pl
from jax.experimental.pallas import tpu as pltpu

_MAXD = 128
_H = 16
_S = 2048
_B = 256
_EXT = 512
_NB = _S // _B


def _rpb_kernel(tab_ref, out_ref, z_ref):
    t_low = tab_ref[0, 0, 0]
    t_high = tab_ref[0, 0, 2 * _MAXD]

    p = jax.lax.broadcasted_iota(jnp.int32, (1, _EXT), 1)
    tabrow = tab_ref[0, 0:1, :]
    big = jnp.concatenate(
        [jnp.full((1, 128), t_low, jnp.float32), tabrow[:, 0:384]], axis=1)
    e2 = jnp.where(p >= 384, t_high, big)
    y = jnp.broadcast_to(e2, (128, _EXT))
    il = jax.lax.broadcasted_iota(jnp.int32, (128, _EXT), 0)
    for b in range(7):
        s = 1 << b
        rolled = jnp.concatenate([y[:, _EXT - s:], y[:, :_EXT - s]], axis=1)
        y = jnp.where((il & s) != 0, rolled, y)
    z_ref[...] = y

    a_m1 = z_ref[:, 128:256]
    a_0 = z_ref[:, 256:384]
    a_p1 = z_ref[:, 384:512]

    l128 = jnp.full((128, 128), t_low, jnp.float32)
    h128 = jnp.full((128, 128), t_high, jnp.float32)
    low_t = jnp.full((_B, _B), t_low, jnp.float32)
    high_t = jnp.full((_B, _B), t_high, jnp.float32)

    t_0 = jnp.concatenate(
        [jnp.concatenate([a_0, a_p1], axis=1),
         jnp.concatenate([a_m1, a_0], axis=1)], axis=0)
    t_p1 = jnp.concatenate(
        [jnp.concatenate([h128, h128], axis=1),
         jnp.concatenate([a_p1, h128], axis=1)], axis=0)
    t_m1 = jnp.concatenate(
        [jnp.concatenate([l128, a_m1], axis=1),
         jnp.concatenate([l128, l128], axis=1)], axis=0)

    for ti in range(_NB):
        for tc in range(_NB):
            dt = tc - ti
            if dt <= -2:
                val = low_t
            elif dt == -1:
                val = t_m1
            elif dt == 0:
                val = t_0
            elif dt == 1:
                val = t_p1
            else:
                val = high_t
            out_ref[0, 0, ti * _B:(ti + 1) * _B, tc * _B:(tc + 1) * _B] = val


def kernel(seq_len, table):
    tableT = jnp.zeros((_H, 1, 512), jnp.float32).at[:, 0, :2 * _MAXD + 1].set(
        table.T.astype(jnp.float32))
    out = pl.pallas_call(
        _rpb_kernel,
        grid=(_H,),
        in_specs=[pl.BlockSpec((1, 1, 512), lambda h: (h, 0, 0))],
        out_specs=pl.BlockSpec((1, 1, _S, _S), lambda h: (0, h, 0, 0)),
        out_shape=jax.ShapeDtypeStruct((1, _H, _S, _S), jnp.float32),
        scratch_shapes=[pltpu.VMEM((128, _EXT), jnp.float32)],
        compiler_params=pltpu.CompilerParams(
            dimension_semantics=("arbitrary",)),
    )(tableT)
    return out
